# baseline (speedup 1.0000x reference)
import functools

import jax
import jax.numpy as jnp
from jax import lax
from jax.experimental import pallas as pl
from jax.experimental.pallas import tpu as pltpu
from jax.experimental.pallas import tpu_sc as plsc

_LANE = 16
_BLK_COLS = 32
_CHUNK = 128
_ROW_BLOCK = 2000


def _sc_segment_sum(table_flat, src2d, dst2d, zeros, cb, n):
    n_chunks = src2d.shape[0]
    info = plsc.get_sparse_core_info()
    ns = info.num_subcores
    cps = n_chunks // ns
    cps_h = cps // 2
    split_last = cb % 2 == 1
    n_part = cb + 1 if split_last else cb
    rpt = ((n + ns - 1) // ns + 7) // 8 * 8

    mesh = plsc.VectorSubcoreMesh(core_axis_name="c", subcore_axis_name="s")

    @functools.partial(
        pl.kernel,
        mesh=mesh,
        out_type=jax.ShapeDtypeStruct((n, n_part * _BLK_COLS), jnp.float32),
        compiler_params=pltpu.CompilerParams(use_tc_tiling_on_sc=False),
        scratch_types=[
            [pltpu.VMEM((_CHUNK,), jnp.int32)] * 2,
            [pltpu.VMEM((_CHUNK,), jnp.int32)] * 2,
            [pltpu.VMEM((_CHUNK,), jnp.int32)] * 2,
            [pltpu.VMEM((_CHUNK, _BLK_COLS), jnp.float32)] * 2,
            pltpu.VMEM_SHARED((n + 8, _BLK_COLS), jnp.float32),
            pltpu.VMEM_SHARED((rpt, _BLK_COLS), jnp.float32),
            [pltpu.SemaphoreType.DMA] * 2,
            [pltpu.SemaphoreType.DMA] * 2,
            [pltpu.SemaphoreType.DMA] * 2,
            pltpu.SemaphoreType.DMA,
        ],
    )
    def seg_kernel(tbl_h, src_h, dst_h, zeros_h, out_h,
                   srcb, dstb, ivb, rowsb, acc_sh, zsh, sem_s, sem_d, sem_g,
                   sem_z):
        core = lax.axis_index("c")
        sub = lax.axis_index("s")
        r0 = pl.multiple_of(jnp.minimum(sub * rpt, n - rpt), 8)

        def load(k0, k, b, lim=None):
            def go():
                pltpu.async_copy(src_h.at[k0 + k], srcb[b], sem_s[b])
                pltpu.async_copy(dst_h.at[k0 + k], dstb[b], sem_d[b])
            if lim is None:
                go()
            else:
                pl.when(k < lim)(go)

        def gather(c, b):
            pltpu.make_async_copy(src_h.at[0], srcb[b], sem_s[b]).wait()
            for j in range(_CHUNK // _LANE):
                sl = pl.ds(j * _LANE, _LANE)
                ivb[b][sl] = srcb[b][sl] * cb + c
            pltpu.async_copy(tbl_h.at[ivb[b]], rowsb[b], sem_g[b])

        def scatter(b):
            pltpu.make_async_copy(tbl_h.at[ivb[b]], rowsb[b], sem_g[b]).wait()
            pltpu.make_async_copy(dst_h.at[0], dstb[b], sem_d[b]).wait()
            pltpu.sync_copy(rowsb[b], acc_sh.at[dstb[b]], add=True)

        def run_block(c, k0, nk, out_col):
            pltpu.async_copy(zsh, acc_sh.at[pl.ds(r0, rpt)], sem_z)

            load(k0, 0, 0)
            gather(c, 0)
            load(k0, 1, 1)

            pltpu.make_async_copy(zsh, acc_sh.at[pl.ds(r0, rpt)], sem_z).wait()
            plsc.subcore_barrier()

            def pair_body(p, carry2):
                k = 2 * p
                scatter(0)
                load(k0, k + 2, 0, lim=nk)
                gather(c, 1)
                scatter(1)
                load(k0, k + 3, 1, lim=nk)
                gather(c, 0)
                return carry2

            lax.fori_loop(0, (nk - 1) // 2, pair_body, 0)
            if nk % 2 == 1:
                scatter(0)
            else:
                gather(c, 1)
                scatter(0)
                scatter(1)

            plsc.subcore_barrier()
            pltpu.sync_copy(acc_sh.at[pl.ds(r0, rpt)],
                            out_h.at[pl.ds(r0, rpt),
                                     pl.ds(pl.multiple_of(out_col, _BLK_COLS),
                                           _BLK_COLS)])
            plsc.subcore_barrier()

        pl.when(sub == 0)(
            lambda: pltpu.sync_copy(zeros_h.at[pl.ds(0, rpt)], zsh))
        plsc.subcore_barrier()

        n_full = cb // 2
        for bi in range(n_full):
            c = bi * 2 + core
            run_block(c, sub * cps, cps, c * _BLK_COLS)
        if split_last:
            c = cb - 1
            run_block(c, sub * cps + core * cps_h, cps_h,
                      (c + core) * _BLK_COLS)

    return seg_kernel(table_flat, src2d, dst2d, zeros)


def _stage_a(xin, aggb, w1_agg, b1, vrow, vw, vb):
    n, din = xin.shape
    emb = w1_agg.shape[1]
    cb = din // _BLK_COLS
    n_agg = aggb.shape[1] // _BLK_COLS
    grid = n // _ROW_BLOCK

    def body(x_ref, a_ref, w1a_ref, b1_ref, v_ref, vw_ref, vb_ref,
             p_ref, s1_ref, s2_ref, hf_ref):
        i = pl.program_id(0)
        s = x_ref[...] + a_ref[:, :din]
        if n_agg == cb + 1:
            last = s[:, din - _BLK_COLS:] + a_ref[:, din:]
            s = jnp.concatenate([s[:, :din - _BLK_COLS], last], axis=1)
        p = jnp.dot(s, w1a_ref[...],
                    preferred_element_type=jnp.float32) + b1_ref[...]
        p_ref[...] = p

        @pl.when(i == 0)
        def _init():
            s1_ref[...] = jnp.zeros_like(s1_ref)
            s2_ref[...] = jnp.zeros_like(s2_ref)
            hf_ref[...] = jnp.dot(v_ref[...], vw_ref[...],
                                  preferred_element_type=jnp.float32) + vb_ref[...]

        s1_ref[...] += jnp.sum(p, axis=0, keepdims=True)
        s2_ref[...] += jnp.sum(p * p, axis=0, keepdims=True)

    dv = vrow.shape[1]
    return pl.pallas_call(
        body,
        grid=(grid,),
        in_specs=[
            pl.BlockSpec((_ROW_BLOCK, din), lambda i: (i, 0)),
            pl.BlockSpec((_ROW_BLOCK, n_agg * _BLK_COLS), lambda i: (i, 0)),
            pl.BlockSpec((cb * _BLK_COLS, emb), lambda i: (0, 0)),
            pl.BlockSpec((1, emb), lambda i: (0, 0)),
            pl.BlockSpec((1, dv), lambda i: (0, 0)),
            pl.BlockSpec((dv, emb), lambda i: (0, 0)),
            pl.BlockSpec((1, emb), lambda i: (0, 0)),
        ],
        out_specs=[
            pl.BlockSpec((_ROW_BLOCK, emb), lambda i: (i, 0)),
            pl.BlockSpec((1, emb), lambda i: (0, 0)),
            pl.BlockSpec((1, emb), lambda i: (0, 0)),
            pl.BlockSpec((1, emb), lambda i: (0, 0)),
        ],
        out_shape=[
            jax.ShapeDtypeStruct((n, emb), jnp.float32),
            jax.ShapeDtypeStruct((1, emb), jnp.float32),
            jax.ShapeDtypeStruct((1, emb), jnp.float32),
            jax.ShapeDtypeStruct((1, emb), jnp.float32),
        ],
    )(xin, aggb, w1_agg, b1, vrow, vw, vb)


def _stage_b(p, s1, s2, g, bb, w2, b2, hf, mut, n_real, final_relu, pad_out):
    n, emb = p.shape
    grid = n // _ROW_BLOCK
    out_cols = emb if not pad_out else ((emb + _BLK_COLS - 1) // _BLK_COLS) * _BLK_COLS

    def body(p_ref, s1_ref, s2_ref, g_ref, bb_ref, w2_ref, b2_ref, hf_ref,
             mut_ref, o_ref):
        i = pl.program_id(0)
        inv_n = 1.0 / float(n_real)
        m = s1_ref[...] * inv_n
        var = s2_ref[...] * inv_n - m * m
        scale = lax.rsqrt(var + 1e-5) * g_ref[...]
        z = (p_ref[...] - m) * scale + bb_ref[...]
        z = jnp.maximum(z, 0.0)
        z = jnp.dot(z, w2_ref[...], preferred_element_type=jnp.float32) + b2_ref[...]
        if final_relu:
            z = jnp.maximum(z, 0.0)
        loc = mut_ref[0] - i * _ROW_BLOCK
        rows = lax.broadcasted_iota(jnp.int32, (_ROW_BLOCK, 1), 0)
        z = z + jnp.where(rows == loc, 1.0, 0.0) * hf_ref[...]
        if pad_out:
            z = jnp.concatenate(
                [z, jnp.zeros((_ROW_BLOCK, out_cols - emb), jnp.float32)], axis=1)
        o_ref[...] = z

    return pl.pallas_call(
        body,
        grid=(grid,),
        in_specs=[
            pl.BlockSpec((_ROW_BLOCK, emb), lambda i: (i, 0)),
            pl.BlockSpec((1, emb), lambda i: (0, 0)),
            pl.BlockSpec((1, emb), lambda i: (0, 0)),
            pl.BlockSpec((1, emb), lambda i: (0, 0)),
            pl.BlockSpec((1, emb), lambda i: (0, 0)),
            pl.BlockSpec((emb, emb), lambda i: (0, 0)),
            pl.BlockSpec((1, emb), lambda i: (0, 0)),
            pl.BlockSpec((1, emb), lambda i: (0, 0)),
            pl.BlockSpec(memory_space=pltpu.SMEM),
        ],
        out_specs=pl.BlockSpec((_ROW_BLOCK, out_cols), lambda i: (i, 0)),
        out_shape=jax.ShapeDtypeStruct((n, out_cols), jnp.float32),
    )(p, s1, s2, g, bb, w2, b2, hf, mut)


def kernel(x, edge_index, mut_res_idx, W1_0, b1_0, g1_0, be1_0, W2_0, b2_0,
           W1_1, b1_1, g1_1, be1_1, W2_1, b2_1, fc1_W, fc1_b, fc2_W, fc2_b):
    n, din = x.shape
    emb = W1_0.shape[1]
    e = edge_index.shape[1]
    mut = mut_res_idx.astype(jnp.int32)

    ns = plsc.get_sparse_core_info().num_subcores
    cps = (e + _CHUNK * ns - 1) // (_CHUNK * ns)
    if cps % 2 == 1:
        cps += 1
    e_pad = cps * ns * _CHUNK
    src = jnp.concatenate(
        [edge_index[0], jnp.zeros((e_pad - e,), jnp.int32)]).reshape(-1, _CHUNK)
    dst = jnp.concatenate(
        [edge_index[1], jnp.full((e_pad - e,), n, jnp.int32)]).reshape(-1, _CHUNK)

    cb0 = (din + _BLK_COLS - 1) // _BLK_COLS
    cb1 = (emb + _BLK_COLS - 1) // _BLK_COLS
    zeros = jnp.zeros(((n // 16 + 7) // 8 * 8, _BLK_COLS), jnp.float32)

    row = lambda v: v.reshape(1, -1)

    xpad = jnp.pad(x, ((0, 0), (0, cb0 * _BLK_COLS - din)))
    agg0 = _sc_segment_sum(xpad.reshape(n * cb0, _BLK_COLS), src, dst, zeros,
                           cb0, n)
    w1a0 = jnp.pad(W1_0, ((0, cb0 * _BLK_COLS - din), (0, 0)))
    xmut = x[mut]
    p0, s1, s2, hf1 = _stage_a(xpad, agg0, w1a0, row(b1_0),
                               xmut, fc1_W, row(fc1_b))
    h1p = _stage_b(p0, s1, s2, row(g1_0), row(be1_0), W2_0, row(b2_0),
                   hf1, mut, n, final_relu=True, pad_out=True)

    agg1 = _sc_segment_sum(h1p.reshape(n * cb1, _BLK_COLS), src, dst, zeros,
                           cb1, n)
    w1p1 = jnp.pad(W1_1, ((0, cb1 * _BLK_COLS - emb), (0, 0)))
    h1mut = h1p[mut][:, :emb]
    p1, t1, t2, hf2 = _stage_a(h1p, agg1, w1p1, row(b1_1),
                               h1mut, fc2_W, row(fc2_b))
    h2 = _stage_b(p1, t1, t2, row(g1_1), row(be1_1), W2_1, row(b2_1),
                  hf2, mut, n, final_relu=False, pad_out=False)
    return h2


# speedup vs baseline: 1.5173x; 1.5173x over previous
import functools

import jax
import jax.numpy as jnp
from jax import lax
from jax.experimental import pallas as pl
from jax.experimental.pallas import tpu as pltpu
from jax.experimental.pallas import tpu_sc as plsc

_LANE = 16
_BLK_COLS = 32
_CHUNK = 128
_ROW_BLOCK = 2000


def _sc_segment_sum(table_flat, src2d, dst2d, zeros, cb, n):
    n_chunks = src2d.shape[0]
    info = plsc.get_sparse_core_info()
    ns = info.num_subcores
    cps = n_chunks // ns
    cps_h = cps // 2
    split_last = cb % 2 == 1
    n_part = cb + 1 if split_last else cb
    rpt = ((n + ns - 1) // ns + 7) // 8 * 8

    mesh = plsc.VectorSubcoreMesh(core_axis_name="c", subcore_axis_name="s")

    @functools.partial(
        pl.kernel,
        mesh=mesh,
        out_type=jax.ShapeDtypeStruct((n, n_part * _BLK_COLS), jnp.float32),
        compiler_params=pltpu.CompilerParams(use_tc_tiling_on_sc=False),
        scratch_types=[
            [pltpu.VMEM((_CHUNK,), jnp.int32)] * 2,
            [pltpu.VMEM((_CHUNK,), jnp.int32)] * 2,
            [pltpu.VMEM((_CHUNK,), jnp.int32)] * 2,
            [pltpu.VMEM((_CHUNK, _BLK_COLS), jnp.float32)] * 2,
            pltpu.VMEM_SHARED((n + 8, _BLK_COLS), jnp.float32),
            [pltpu.SemaphoreType.DMA] * 2,
            [pltpu.SemaphoreType.DMA] * 2,
            [pltpu.SemaphoreType.DMA] * 2,
            pltpu.SemaphoreType.DMA,
        ],
    )
    def seg_kernel(tbl_h, src_h, dst_h, zeros_h, out_h,
                   srcb, dstb, ivb, rowsb, acc_sh, sem_s, sem_d, sem_g,
                   sem_z):
        core = lax.axis_index("c")
        sub = lax.axis_index("s")
        r0 = pl.multiple_of(jnp.minimum(sub * rpt, n - rpt), 8)

        def load(k0, k, b, lim=None):
            def go():
                pltpu.async_copy(src_h.at[k0 + k], srcb[b], sem_s[b])
                pltpu.async_copy(dst_h.at[k0 + k], dstb[b], sem_d[b])
            if lim is None:
                go()
            else:
                pl.when(k < lim)(go)

        def gather(c, b):
            pltpu.make_async_copy(src_h.at[0], srcb[b], sem_s[b]).wait()
            for j in range(_CHUNK // _LANE):
                sl = pl.ds(j * _LANE, _LANE)
                ivb[b][sl] = srcb[b][sl] * cb + c
            pltpu.async_copy(tbl_h.at[ivb[b]], rowsb[b], sem_g[b])

        def scatter(b):
            pltpu.make_async_copy(tbl_h.at[ivb[b]], rowsb[b], sem_g[b]).wait()
            pltpu.make_async_copy(dst_h.at[0], dstb[b], sem_d[b]).wait()
            pltpu.sync_copy(rowsb[b], acc_sh.at[dstb[b]], add=True)

        def run_block(c, k0, nk, out_col):
            pltpu.async_copy(zeros_h.at[pl.ds(0, rpt)],
                             acc_sh.at[pl.ds(r0, rpt)], sem_z)

            load(k0, 0, 0)
            gather(c, 0)
            load(k0, 1, 1)

            pltpu.make_async_copy(zeros_h.at[pl.ds(0, rpt)],
                                  acc_sh.at[pl.ds(r0, rpt)], sem_z).wait()
            plsc.subcore_barrier()

            def pair_body(p, carry2):
                k = 2 * p
                scatter(0)
                load(k0, k + 2, 0, lim=nk)
                gather(c, 1)
                scatter(1)
                load(k0, k + 3, 1, lim=nk)
                gather(c, 0)
                return carry2

            lax.fori_loop(0, (nk - 1) // 2, pair_body, 0)
            if nk % 2 == 1:
                scatter(0)
            else:
                gather(c, 1)
                scatter(0)
                scatter(1)

            plsc.subcore_barrier()
            pltpu.sync_copy(acc_sh.at[pl.ds(r0, rpt)],
                            out_h.at[pl.ds(r0, rpt),
                                     pl.ds(pl.multiple_of(out_col, _BLK_COLS),
                                           _BLK_COLS)])
            plsc.subcore_barrier()

        n_full = cb // 2
        for bi in range(n_full):
            c = bi * 2 + core
            run_block(c, sub * cps, cps, c * _BLK_COLS)
        if split_last:
            c = cb - 1
            run_block(c, sub * cps + core * cps_h, cps_h,
                      (c + core) * _BLK_COLS)

    return seg_kernel(table_flat, src2d, dst2d, zeros)


def _stage_a(xin, aggb, w1_agg, b1, vrow, vw, vb):
    n, din = xin.shape
    emb = w1_agg.shape[1]
    cb = din // _BLK_COLS
    n_agg = aggb.shape[1] // _BLK_COLS
    grid = n // _ROW_BLOCK

    def body(x_ref, a_ref, w1a_ref, b1_ref, v_ref, vw_ref, vb_ref,
             p_ref, s1_ref, s2_ref, hf_ref):
        i = pl.program_id(0)
        s = x_ref[...] + a_ref[:, :din]
        if n_agg == cb + 1:
            last = s[:, din - _BLK_COLS:] + a_ref[:, din:]
            s = jnp.concatenate([s[:, :din - _BLK_COLS], last], axis=1)
        p = jnp.dot(s, w1a_ref[...],
                    preferred_element_type=jnp.float32) + b1_ref[...]
        p_ref[...] = p

        @pl.when(i == 0)
        def _init():
            s1_ref[...] = jnp.zeros_like(s1_ref)
            s2_ref[...] = jnp.zeros_like(s2_ref)
            hf_ref[...] = jnp.dot(v_ref[...], vw_ref[...],
                                  preferred_element_type=jnp.float32) + vb_ref[...]

        s1_ref[...] += jnp.sum(p, axis=0, keepdims=True)
        s2_ref[...] += jnp.sum(p * p, axis=0, keepdims=True)

    dv = vrow.shape[1]
    return pl.pallas_call(
        body,
        grid=(grid,),
        in_specs=[
            pl.BlockSpec((_ROW_BLOCK, din), lambda i: (i, 0)),
            pl.BlockSpec((_ROW_BLOCK, n_agg * _BLK_COLS), lambda i: (i, 0)),
            pl.BlockSpec((cb * _BLK_COLS, emb), lambda i: (0, 0)),
            pl.BlockSpec((1, emb), lambda i: (0, 0)),
            pl.BlockSpec((1, dv), lambda i: (0, 0)),
            pl.BlockSpec((dv, emb), lambda i: (0, 0)),
            pl.BlockSpec((1, emb), lambda i: (0, 0)),
        ],
        out_specs=[
            pl.BlockSpec((_ROW_BLOCK, emb), lambda i: (i, 0)),
            pl.BlockSpec((1, emb), lambda i: (0, 0)),
            pl.BlockSpec((1, emb), lambda i: (0, 0)),
            pl.BlockSpec((1, emb), lambda i: (0, 0)),
        ],
        out_shape=[
            jax.ShapeDtypeStruct((n, emb), jnp.float32),
            jax.ShapeDtypeStruct((1, emb), jnp.float32),
            jax.ShapeDtypeStruct((1, emb), jnp.float32),
            jax.ShapeDtypeStruct((1, emb), jnp.float32),
        ],
    )(xin, aggb, w1_agg, b1, vrow, vw, vb)


def _stage_b(p, s1, s2, g, bb, w2, b2, hf, mut, n_real, final_relu, pad_out):
    n, emb = p.shape
    grid = n // _ROW_BLOCK
    out_cols = emb if not pad_out else ((emb + _BLK_COLS - 1) // _BLK_COLS) * _BLK_COLS

    def body(p_ref, s1_ref, s2_ref, g_ref, bb_ref, w2_ref, b2_ref, hf_ref,
             mut_ref, o_ref):
        i = pl.program_id(0)
        inv_n = 1.0 / float(n_real)
        m = s1_ref[...] * inv_n
        var = s2_ref[...] * inv_n - m * m
        scale = lax.rsqrt(var + 1e-5) * g_ref[...]
        z = (p_ref[...] - m) * scale + bb_ref[...]
        z = jnp.maximum(z, 0.0)
        z = jnp.dot(z, w2_ref[...], preferred_element_type=jnp.float32) + b2_ref[...]
        if final_relu:
            z = jnp.maximum(z, 0.0)
        loc = mut_ref[0] - i * _ROW_BLOCK
        rows = lax.broadcasted_iota(jnp.int32, (_ROW_BLOCK, 1), 0)
        z = z + jnp.where(rows == loc, 1.0, 0.0) * hf_ref[...]
        if pad_out:
            z = jnp.concatenate(
                [z, jnp.zeros((_ROW_BLOCK, out_cols - emb), jnp.float32)], axis=1)
        o_ref[...] = z

    return pl.pallas_call(
        body,
        grid=(grid,),
        in_specs=[
            pl.BlockSpec((_ROW_BLOCK, emb), lambda i: (i, 0)),
            pl.BlockSpec((1, emb), lambda i: (0, 0)),
            pl.BlockSpec((1, emb), lambda i: (0, 0)),
            pl.BlockSpec((1, emb), lambda i: (0, 0)),
            pl.BlockSpec((1, emb), lambda i: (0, 0)),
            pl.BlockSpec((emb, emb), lambda i: (0, 0)),
            pl.BlockSpec((1, emb), lambda i: (0, 0)),
            pl.BlockSpec((1, emb), lambda i: (0, 0)),
            pl.BlockSpec(memory_space=pltpu.SMEM),
        ],
        out_specs=pl.BlockSpec((_ROW_BLOCK, out_cols), lambda i: (i, 0)),
        out_shape=jax.ShapeDtypeStruct((n, out_cols), jnp.float32),
    )(p, s1, s2, g, bb, w2, b2, hf, mut)


def kernel(x, edge_index, mut_res_idx, W1_0, b1_0, g1_0, be1_0, W2_0, b2_0,
           W1_1, b1_1, g1_1, be1_1, W2_1, b2_1, fc1_W, fc1_b, fc2_W, fc2_b):
    n, din = x.shape
    emb = W1_0.shape[1]
    e = edge_index.shape[1]
    mut = mut_res_idx.astype(jnp.int32)

    ns = plsc.get_sparse_core_info().num_subcores
    cps = (e + _CHUNK * ns - 1) // (_CHUNK * ns)
    if cps % 2 == 1:
        cps += 1
    e_pad = cps * ns * _CHUNK
    src = jnp.concatenate(
        [edge_index[0], jnp.zeros((e_pad - e,), jnp.int32)]).reshape(-1, _CHUNK)
    dst = jnp.concatenate(
        [edge_index[1], jnp.full((e_pad - e,), n, jnp.int32)]).reshape(-1, _CHUNK)

    cb0 = (din + _BLK_COLS - 1) // _BLK_COLS
    cb1 = (emb + _BLK_COLS - 1) // _BLK_COLS
    zeros = jnp.zeros(((n // 16 + 7) // 8 * 8, _BLK_COLS), jnp.float32)

    row = lambda v: v.reshape(1, -1)

    xpad = jnp.pad(x, ((0, 0), (0, cb0 * _BLK_COLS - din)))
    agg0 = _sc_segment_sum(xpad.reshape(n * cb0, _BLK_COLS), src, dst, zeros,
                           cb0, n)
    w1a0 = jnp.pad(W1_0, ((0, cb0 * _BLK_COLS - din), (0, 0)))
    xmut = x[mut]
    p0, s1, s2, hf1 = _stage_a(xpad, agg0, w1a0, row(b1_0),
                               xmut, fc1_W, row(fc1_b))
    h1p = _stage_b(p0, s1, s2, row(g1_0), row(be1_0), W2_0, row(b2_0),
                   hf1, mut, n, final_relu=True, pad_out=True)

    agg1 = _sc_segment_sum(h1p.reshape(n * cb1, _BLK_COLS), src, dst, zeros,
                           cb1, n)
    w1p1 = jnp.pad(W1_1, ((0, cb1 * _BLK_COLS - emb), (0, 0)))
    h1mut = h1p[mut][:, :emb]
    p1, t1, t2, hf2 = _stage_a(h1p, agg1, w1p1, row(b1_1),
                               h1mut, fc2_W, row(fc2_b))
    h2 = _stage_b(p1, t1, t2, row(g1_1), row(be1_1), W2_1, row(b2_1),
                  hf2, mut, n, final_relu=False, pad_out=False)
    return h2


# speedup vs baseline: 1.5358x; 1.0122x over previous
import functools

import jax
import jax.numpy as jnp
from jax import lax
from jax.experimental import pallas as pl
from jax.experimental.pallas import tpu as pltpu
from jax.experimental.pallas import tpu_sc as plsc

_LANE = 16
_BLK_COLS = 32
_CHUNK = 128
_ROW_BLOCK = 5000


def _sc_segment_sum(table_flat, src2d, dst2d, zeros, cb, n):
    n_chunks = src2d.shape[0]
    info = plsc.get_sparse_core_info()
    ns = info.num_subcores
    cps = n_chunks // ns
    cps_h = cps // 2
    split_last = cb % 2 == 1
    n_part = cb + 1 if split_last else cb
    rpt = ((n + ns - 1) // ns + 7) // 8 * 8

    mesh = plsc.VectorSubcoreMesh(core_axis_name="c", subcore_axis_name="s")

    @functools.partial(
        pl.kernel,
        mesh=mesh,
        out_type=jax.ShapeDtypeStruct((n, n_part * _BLK_COLS), jnp.float32),
        compiler_params=pltpu.CompilerParams(use_tc_tiling_on_sc=False),
        scratch_types=[
            [pltpu.VMEM((_CHUNK,), jnp.int32)] * 2,
            [pltpu.VMEM((_CHUNK,), jnp.int32)] * 2,
            [pltpu.VMEM((_CHUNK,), jnp.int32)] * 2,
            [pltpu.VMEM((_CHUNK, _BLK_COLS), jnp.float32)] * 2,
            pltpu.VMEM_SHARED((n + 8, _BLK_COLS), jnp.float32),
            [pltpu.SemaphoreType.DMA] * 2,
            [pltpu.SemaphoreType.DMA] * 2,
            [pltpu.SemaphoreType.DMA] * 2,
            pltpu.SemaphoreType.DMA,
        ],
    )
    def seg_kernel(tbl_h, src_h, dst_h, zeros_h, out_h,
                   srcb, dstb, ivb, rowsb, acc_sh, sem_s, sem_d, sem_g,
                   sem_z):
        core = lax.axis_index("c")
        sub = lax.axis_index("s")
        r0 = pl.multiple_of(jnp.minimum(sub * rpt, n - rpt), 8)

        def load(k0, k, b, lim=None):
            def go():
                pltpu.async_copy(src_h.at[k0 + k], srcb[b], sem_s[b])
                pltpu.async_copy(dst_h.at[k0 + k], dstb[b], sem_d[b])
            if lim is None:
                go()
            else:
                pl.when(k < lim)(go)

        def gather(c, b):
            pltpu.make_async_copy(src_h.at[0], srcb[b], sem_s[b]).wait()
            for j in range(_CHUNK // _LANE):
                sl = pl.ds(j * _LANE, _LANE)
                ivb[b][sl] = srcb[b][sl] * cb + c
            pltpu.async_copy(tbl_h.at[ivb[b]], rowsb[b], sem_g[b])

        def scatter(b):
            pltpu.make_async_copy(tbl_h.at[ivb[b]], rowsb[b], sem_g[b]).wait()
            pltpu.make_async_copy(dst_h.at[0], dstb[b], sem_d[b]).wait()
            pltpu.sync_copy(rowsb[b], acc_sh.at[dstb[b]], add=True)

        def run_block(c, k0, nk, out_col):
            pltpu.async_copy(zeros_h.at[pl.ds(0, rpt)],
                             acc_sh.at[pl.ds(r0, rpt)], sem_z)

            load(k0, 0, 0)
            gather(c, 0)
            load(k0, 1, 1)

            pltpu.make_async_copy(zeros_h.at[pl.ds(0, rpt)],
                                  acc_sh.at[pl.ds(r0, rpt)], sem_z).wait()
            plsc.subcore_barrier()

            def pair_body(p, carry2):
                k = 2 * p
                scatter(0)
                load(k0, k + 2, 0, lim=nk)
                gather(c, 1)
                scatter(1)
                load(k0, k + 3, 1, lim=nk)
                gather(c, 0)
                return carry2

            lax.fori_loop(0, (nk - 1) // 2, pair_body, 0)
            if nk % 2 == 1:
                scatter(0)
            else:
                gather(c, 1)
                scatter(0)
                scatter(1)

            plsc.subcore_barrier()
            pltpu.sync_copy(acc_sh.at[pl.ds(r0, rpt)],
                            out_h.at[pl.ds(r0, rpt),
                                     pl.ds(pl.multiple_of(out_col, _BLK_COLS),
                                           _BLK_COLS)])
            plsc.subcore_barrier()

        n_full = cb // 2
        for bi in range(n_full):
            c = bi * 2 + core
            run_block(c, sub * cps, cps, c * _BLK_COLS)
        if split_last:
            c = cb - 1
            run_block(c, sub * cps + core * cps_h, cps_h,
                      (c + core) * _BLK_COLS)

    return seg_kernel(table_flat, src2d, dst2d, zeros)


def _stage_a(xin, aggb, w1_agg, b1, vrow, vw, vb):
    n, din = xin.shape
    emb = w1_agg.shape[1]
    cb = din // _BLK_COLS
    n_agg = aggb.shape[1] // _BLK_COLS
    grid = n // _ROW_BLOCK

    def body(x_ref, a_ref, w1a_ref, b1_ref, v_ref, vw_ref, vb_ref,
             p_ref, s1_ref, s2_ref, hf_ref):
        i = pl.program_id(0)
        s = x_ref[...] + a_ref[:, :din]
        if n_agg == cb + 1:
            last = s[:, din - _BLK_COLS:] + a_ref[:, din:]
            s = jnp.concatenate([s[:, :din - _BLK_COLS], last], axis=1)
        p = jnp.dot(s, w1a_ref[...],
                    preferred_element_type=jnp.float32) + b1_ref[...]
        p_ref[...] = p

        @pl.when(i == 0)
        def _init():
            s1_ref[...] = jnp.zeros_like(s1_ref)
            s2_ref[...] = jnp.zeros_like(s2_ref)
            hf_ref[...] = jnp.dot(v_ref[...], vw_ref[...],
                                  preferred_element_type=jnp.float32) + vb_ref[...]

        s1_ref[...] += jnp.sum(p, axis=0, keepdims=True)
        s2_ref[...] += jnp.sum(p * p, axis=0, keepdims=True)

    dv = vrow.shape[1]
    return pl.pallas_call(
        body,
        grid=(grid,),
        in_specs=[
            pl.BlockSpec((_ROW_BLOCK, din), lambda i: (i, 0)),
            pl.BlockSpec((_ROW_BLOCK, n_agg * _BLK_COLS), lambda i: (i, 0)),
            pl.BlockSpec((cb * _BLK_COLS, emb), lambda i: (0, 0)),
            pl.BlockSpec((1, emb), lambda i: (0, 0)),
            pl.BlockSpec((1, dv), lambda i: (0, 0)),
            pl.BlockSpec((dv, emb), lambda i: (0, 0)),
            pl.BlockSpec((1, emb), lambda i: (0, 0)),
        ],
        out_specs=[
            pl.BlockSpec((_ROW_BLOCK, emb), lambda i: (i, 0)),
            pl.BlockSpec((1, emb), lambda i: (0, 0)),
            pl.BlockSpec((1, emb), lambda i: (0, 0)),
            pl.BlockSpec((1, emb), lambda i: (0, 0)),
        ],
        out_shape=[
            jax.ShapeDtypeStruct((n, emb), jnp.float32),
            jax.ShapeDtypeStruct((1, emb), jnp.float32),
            jax.ShapeDtypeStruct((1, emb), jnp.float32),
            jax.ShapeDtypeStruct((1, emb), jnp.float32),
        ],
    )(xin, aggb, w1_agg, b1, vrow, vw, vb)


def _stage_b(p, s1, s2, g, bb, w2, b2, hf, mut, n_real, final_relu, pad_out):
    n, emb = p.shape
    grid = n // _ROW_BLOCK
    out_cols = emb if not pad_out else ((emb + _BLK_COLS - 1) // _BLK_COLS) * _BLK_COLS

    def body(p_ref, s1_ref, s2_ref, g_ref, bb_ref, w2_ref, b2_ref, hf_ref,
             mut_ref, o_ref):
        i = pl.program_id(0)
        inv_n = 1.0 / float(n_real)
        m = s1_ref[...] * inv_n
        var = s2_ref[...] * inv_n - m * m
        scale = lax.rsqrt(var + 1e-5) * g_ref[...]
        z = (p_ref[...] - m) * scale + bb_ref[...]
        z = jnp.maximum(z, 0.0)
        z = jnp.dot(z, w2_ref[...], preferred_element_type=jnp.float32) + b2_ref[...]
        if final_relu:
            z = jnp.maximum(z, 0.0)
        loc = mut_ref[0] - i * _ROW_BLOCK
        rows = lax.broadcasted_iota(jnp.int32, (_ROW_BLOCK, 1), 0)
        z = z + jnp.where(rows == loc, 1.0, 0.0) * hf_ref[...]
        if pad_out:
            z = jnp.concatenate(
                [z, jnp.zeros((_ROW_BLOCK, out_cols - emb), jnp.float32)], axis=1)
        o_ref[...] = z

    return pl.pallas_call(
        body,
        grid=(grid,),
        in_specs=[
            pl.BlockSpec((_ROW_BLOCK, emb), lambda i: (i, 0)),
            pl.BlockSpec((1, emb), lambda i: (0, 0)),
            pl.BlockSpec((1, emb), lambda i: (0, 0)),
            pl.BlockSpec((1, emb), lambda i: (0, 0)),
            pl.BlockSpec((1, emb), lambda i: (0, 0)),
            pl.BlockSpec((emb, emb), lambda i: (0, 0)),
            pl.BlockSpec((1, emb), lambda i: (0, 0)),
            pl.BlockSpec((1, emb), lambda i: (0, 0)),
            pl.BlockSpec(memory_space=pltpu.SMEM),
        ],
        out_specs=pl.BlockSpec((_ROW_BLOCK, out_cols), lambda i: (i, 0)),
        out_shape=jax.ShapeDtypeStruct((n, out_cols), jnp.float32),
    )(p, s1, s2, g, bb, w2, b2, hf, mut)


def kernel(x, edge_index, mut_res_idx, W1_0, b1_0, g1_0, be1_0, W2_0, b2_0,
           W1_1, b1_1, g1_1, be1_1, W2_1, b2_1, fc1_W, fc1_b, fc2_W, fc2_b):
    n, din = x.shape
    emb = W1_0.shape[1]
    e = edge_index.shape[1]
    mut = mut_res_idx.astype(jnp.int32)

    ns = plsc.get_sparse_core_info().num_subcores
    cps = (e + _CHUNK * ns - 1) // (_CHUNK * ns)
    if cps % 2 == 1:
        cps += 1
    e_pad = cps * ns * _CHUNK
    src = jnp.concatenate(
        [edge_index[0], jnp.zeros((e_pad - e,), jnp.int32)]).reshape(-1, _CHUNK)
    dst = jnp.concatenate(
        [edge_index[1], jnp.full((e_pad - e,), n, jnp.int32)]).reshape(-1, _CHUNK)

    cb0 = (din + _BLK_COLS - 1) // _BLK_COLS
    cb1 = (emb + _BLK_COLS - 1) // _BLK_COLS
    zeros = jnp.zeros(((n // 16 + 7) // 8 * 8, _BLK_COLS), jnp.float32)

    row = lambda v: v.reshape(1, -1)

    xpad = jnp.pad(x, ((0, 0), (0, cb0 * _BLK_COLS - din)))
    agg0 = _sc_segment_sum(xpad.reshape(n * cb0, _BLK_COLS), src, dst, zeros,
                           cb0, n)
    w1a0 = jnp.pad(W1_0, ((0, cb0 * _BLK_COLS - din), (0, 0)))
    xmut = x[mut]
    p0, s1, s2, hf1 = _stage_a(xpad, agg0, w1a0, row(b1_0),
                               xmut, fc1_W, row(fc1_b))
    h1p = _stage_b(p0, s1, s2, row(g1_0), row(be1_0), W2_0, row(b2_0),
                   hf1, mut, n, final_relu=True, pad_out=True)

    agg1 = _sc_segment_sum(h1p.reshape(n * cb1, _BLK_COLS), src, dst, zeros,
                           cb1, n)
    w1p1 = jnp.pad(W1_1, ((0, cb1 * _BLK_COLS - emb), (0, 0)))
    h1mut = h1p[mut][:, :emb]
    p1, t1, t2, hf2 = _stage_a(h1p, agg1, w1p1, row(b1_1),
                               h1mut, fc2_W, row(fc2_b))
    h2 = _stage_b(p1, t1, t2, row(g1_1), row(be1_1), W2_1, row(b2_1),
                  hf2, mut, n, final_relu=False, pad_out=False)
    return h2
